# Initial kernel scaffold; baseline (speedup 1.0000x reference)
import jax
import jax.numpy as jnp
from jax.experimental import pallas as pl


def kernel(x, edge_attr, edge_W0, edge_b0, edge_W1, edge_b1, edge_W2, edge_b2, edge_ln_g, edge_ln_b, node_W0, node_b0, node_W1, node_b1, node_W2, node_b2, node_ln_g, node_ln_b, edge_index):
    raise NotImplementedError("write your pallas kernel here")


# speedup vs baseline: 2.8187x; 2.8187x over previous
import functools

import jax
import jax.numpy as jnp
from jax import lax
from jax.experimental import pallas as pl
from jax.experimental.pallas import tpu as pltpu, tpu_sc as plsc

N = 10000
E = 320000
D = 128
H = 128

NC = 2
NS = 16
NW = NC * NS
EP = E // NW
CH = 80
NCHUNK = EP // CH
RSTEP = 624
RP = 640

_F32 = jnp.float32


def _dot(a, b):
    return jnp.dot(a, b, preferred_element_type=jnp.float32)


def _pre_body(x_ref, w_ref, p_ref, q_ref):
    xv = x_ref[...]
    p_ref[...] = _dot(xv, w_ref[0:D, :])
    q_ref[...] = _dot(xv, w_ref[D:2 * D, :])


def _pre(x, w0ab, block):
    grid = N // block
    return pl.pallas_call(
        _pre_body,
        grid=(grid,),
        in_specs=[
            pl.BlockSpec((block, D), lambda i: (i, 0)),
            pl.BlockSpec((2 * D, H), lambda i: (0, 0)),
        ],
        out_specs=[
            pl.BlockSpec((block, H), lambda i: (i, 0)),
            pl.BlockSpec((block, H), lambda i: (i, 0)),
        ],
        out_shape=[
            jax.ShapeDtypeStruct((N, H), _F32),
            jax.ShapeDtypeStruct((N, H), _F32),
        ],
    )(x, w0ab)


def _ln(h, g, beta):
    mu = jnp.mean(h, axis=-1, keepdims=True)
    d = h - mu
    var = jnp.mean(d * d, axis=-1, keepdims=True)
    return d * lax.rsqrt(var + 1e-5) * g + beta


def _edge_body(gp_ref, gq_ref, ea_ref, w0c_ref, b0_ref, w1_ref, b1_ref,
               w2_ref, b2_ref, g_ref, beta_ref, out_ref):
    ea = ea_ref[...]
    h = gp_ref[...] + gq_ref[...] + _dot(ea, w0c_ref[...]) + b0_ref[...]
    h = jnp.maximum(h, 0.0)
    h = jnp.maximum(_dot(h, w1_ref[...]) + b1_ref[...], 0.0)
    h = _dot(h, w2_ref[...]) + b2_ref[...]
    out_ref[...] = _ln(h, g_ref[...], beta_ref[...]) + ea


def _edge_mlp(gp, gq, ea, w0c, b0, w1, b1, w2, b2, g, beta, block):
    grid = E // block
    row_spec = pl.BlockSpec((block, D), lambda i: (i, 0))
    w_spec = pl.BlockSpec((H, H), lambda i: (0, 0))
    v_spec = pl.BlockSpec((1, H), lambda i: (0, 0))
    return pl.pallas_call(
        _edge_body,
        grid=(grid,),
        in_specs=[row_spec, row_spec, row_spec,
                  pl.BlockSpec((H, H), lambda i: (0, 0)), v_spec,
                  w_spec, v_spec, pl.BlockSpec((H, D), lambda i: (0, 0)),
                  v_spec, v_spec, v_spec],
        out_specs=pl.BlockSpec((block, D), lambda i: (i, 0)),
        out_shape=jax.ShapeDtypeStruct((E, D), _F32),
    )(gp, gq, ea, w0c, b0, w1, b1, w2, b2, g, beta)


def _node_body(x_ref, a0_ref, a1_ref, w0a_ref, w0b_ref, b0_ref, w1_ref,
               b1_ref, w2_ref, b2_ref, g_ref, beta_ref, out_ref):
    xv = x_ref[...]
    agg = a0_ref[...] + a1_ref[...]
    h = _dot(xv, w0a_ref[...]) + _dot(agg, w0b_ref[...]) + b0_ref[...]
    h = jnp.maximum(h, 0.0)
    h = jnp.maximum(_dot(h, w1_ref[...]) + b1_ref[...], 0.0)
    h = _dot(h, w2_ref[...]) + b2_ref[...]
    out_ref[...] = _ln(h, g_ref[...], beta_ref[...]) + xv


def _node_mlp(x, a0, a1, w0a, w0b, b0, w1, b1, w2, b2, g, beta, block):
    grid = N // block
    row_spec = pl.BlockSpec((block, D), lambda i: (i, 0))
    w_spec = pl.BlockSpec((H, H), lambda i: (0, 0))
    v_spec = pl.BlockSpec((1, H), lambda i: (0, 0))
    return pl.pallas_call(
        _node_body,
        grid=(grid,),
        in_specs=[row_spec, row_spec, row_spec,
                  w_spec, w_spec, v_spec, w_spec, v_spec,
                  pl.BlockSpec((H, D), lambda i: (0, 0)), v_spec, v_spec,
                  v_spec],
        out_specs=pl.BlockSpec((block, D), lambda i: (i, 0)),
        out_shape=jax.ShapeDtypeStruct((N, D), _F32),
    )(x, a0, a1, w0a, w0b, b0, w1, b1, w2, b2, g, beta)


_MESH = plsc.VectorSubcoreMesh(core_axis_name="c", subcore_axis_name="s",
                               num_cores=NC, num_subcores=NS)


@functools.partial(
    pl.kernel,
    out_type=[jax.ShapeDtypeStruct((E, H), _F32),
              jax.ShapeDtypeStruct((E, H), _F32)],
    mesh=_MESH,
    scratch_types=[
        pltpu.VMEM((CH,), jnp.int32),
        pltpu.VMEM((CH,), jnp.int32),
        pltpu.VMEM((CH, H), _F32),
        pltpu.VMEM((CH, H), _F32),
        pltpu.SemaphoreType.DMA,
        pltpu.SemaphoreType.DMA,
    ],
)
def _sc_gather(p_hbm, q_hbm, row_hbm, col_hbm, gp_hbm, gq_hbm,
               idxr, idxc, bufp, bufq, semp, semq):
    wid = lax.axis_index("s") * NC + lax.axis_index("c")
    base = wid * EP

    def body(t, _):
        off = base + t * CH
        pltpu.sync_copy(row_hbm.at[pl.ds(off, CH)], idxr)
        pltpu.sync_copy(col_hbm.at[pl.ds(off, CH)], idxc)
        cp = pltpu.async_copy(p_hbm.at[idxr], bufp, semp)
        cq = pltpu.async_copy(q_hbm.at[idxc], bufq, semq)
        cp.wait()
        cq.wait()
        pltpu.sync_copy(bufp, gp_hbm.at[pl.ds(off, CH)])
        pltpu.sync_copy(bufq, gq_hbm.at[pl.ds(off, CH)])
        return _

    lax.fori_loop(0, NCHUNK, body, None)


@functools.partial(
    pl.kernel,
    out_type=jax.ShapeDtypeStruct((NC, N, H), _F32),
    mesh=_MESH,
    scratch_types=[
        pltpu.VMEM((1, CH), jnp.int32),
        pltpu.VMEM((CH, H), _F32),
        pltpu.VMEM_SHARED((N, H), _F32),
    ],
)
def _sc_scatter(ea_hbm, col_hbm, zeros_hbm, out_hbm, idx, buf, acc):
    cid = lax.axis_index("c")
    sid = lax.axis_index("s")
    wid = sid * NC + cid
    base = wid * EP

    def zbody(k, _):
        r = sid * RSTEP + k * CH
        pltpu.sync_copy(zeros_hbm.at[pl.ds(r, CH)], buf)
        pltpu.sync_copy(buf, acc.at[pl.ds(r, CH)])
        return _

    lax.fori_loop(0, RP // CH, zbody, None)
    plsc.subcore_barrier()

    def body(t, _):
        off = base + t * CH
        pltpu.sync_copy(col_hbm.at[pl.ds(off, CH)], idx.at[0])
        pltpu.sync_copy(ea_hbm.at[pl.ds(off, CH)], buf)
        pltpu.sync_copy(buf, acc.at[idx.at[0]], add=True)
        return _

    lax.fori_loop(0, NCHUNK, body, None)
    plsc.subcore_barrier()

    def obody(k, _):
        r = sid * RSTEP + k * CH
        pltpu.sync_copy(acc.at[pl.ds(r, CH)], buf)
        pltpu.sync_copy(buf, out_hbm.at[cid, pl.ds(r, CH)])
        return _

    lax.fori_loop(0, RP // CH, obody, None)


def kernel(x, edge_attr, edge_W0, edge_b0, edge_W1, edge_b1, edge_W2,
           edge_b2, edge_ln_g, edge_ln_b, node_W0, node_b0, node_W1,
           node_b1, node_W2, node_b2, node_ln_g, node_ln_b, edge_index):
    row = edge_index[0]
    col = edge_index[1]
    zeros = jnp.zeros((N, H), _F32)
    nb = edge_W0.shape[0]

    def v(a):
        return a.reshape(1, -1)

    for b in range(nb):
        p, q = _pre(x, edge_W0[b], block=2000)
        gp, gq = _sc_gather(p, q, row, col)
        edge_attr = _edge_mlp(gp, gq, edge_attr, edge_W0[b][2 * D:], v(edge_b0[b]),
                              edge_W1[b], v(edge_b1[b]), edge_W2[b], v(edge_b2[b]),
                              v(edge_ln_g[b]), v(edge_ln_b[b]), block=2560)
        agg = _sc_scatter(edge_attr, col, zeros)
        x = _node_mlp(x, agg[0], agg[1], node_W0[b][:D], node_W0[b][D:],
                      v(node_b0[b]), node_W1[b], v(node_b1[b]), node_W2[b],
                      v(node_b2[b]), v(node_ln_g[b]), v(node_ln_b[b]),
                      block=2000)
    return (x, edge_attr)


# speedup vs baseline: 3.8426x; 1.3633x over previous
import functools

import jax
import jax.numpy as jnp
from jax import lax
from jax.experimental import pallas as pl
from jax.experimental.pallas import tpu as pltpu, tpu_sc as plsc

N = 10000
E = 320000
D = 128
H = 128

NC = 2
NS = 16
NW = NC * NS
EP = E // NW
CH = 80
NCHUNK = EP // CH
RSTEP = 624
RP = 640

_F32 = jnp.float32


def _dot(a, b):
    return jnp.dot(a, b, preferred_element_type=jnp.float32)


def _pre_body(x_ref, w_ref, p_ref, q_ref):
    xv = x_ref[...]
    p_ref[...] = _dot(xv, w_ref[0:D, :])
    q_ref[...] = _dot(xv, w_ref[D:2 * D, :])


def _pre(x, w0ab, block):
    grid = N // block
    return pl.pallas_call(
        _pre_body,
        grid=(grid,),
        in_specs=[
            pl.BlockSpec((block, D), lambda i: (i, 0)),
            pl.BlockSpec((2 * D, H), lambda i: (0, 0)),
        ],
        out_specs=[
            pl.BlockSpec((block, H), lambda i: (i, 0)),
            pl.BlockSpec((block, H), lambda i: (i, 0)),
        ],
        out_shape=[
            jax.ShapeDtypeStruct((N, H), _F32),
            jax.ShapeDtypeStruct((N, H), _F32),
        ],
    )(x, w0ab)


def _ln(h, g, beta):
    mu = jnp.mean(h, axis=-1, keepdims=True)
    d = h - mu
    var = jnp.mean(d * d, axis=-1, keepdims=True)
    return d * lax.rsqrt(var + 1e-5) * g + beta


def _edge_body(gp_ref, gq_ref, ea_ref, w0c_ref, b0_ref, w1_ref, b1_ref,
               w2_ref, b2_ref, g_ref, beta_ref, out_ref):
    ea = ea_ref[...]
    h = gp_ref[...] + gq_ref[...] + _dot(ea, w0c_ref[...]) + b0_ref[...]
    h = jnp.maximum(h, 0.0)
    h = jnp.maximum(_dot(h, w1_ref[...]) + b1_ref[...], 0.0)
    h = _dot(h, w2_ref[...]) + b2_ref[...]
    out_ref[...] = _ln(h, g_ref[...], beta_ref[...]) + ea


def _edge_mlp(gp, gq, ea, w0c, b0, w1, b1, w2, b2, g, beta, block):
    grid = E // block
    row_spec = pl.BlockSpec((block, D), lambda i: (i, 0))
    w_spec = pl.BlockSpec((H, H), lambda i: (0, 0))
    v_spec = pl.BlockSpec((1, H), lambda i: (0, 0))
    return pl.pallas_call(
        _edge_body,
        grid=(grid,),
        in_specs=[row_spec, row_spec, row_spec,
                  pl.BlockSpec((H, H), lambda i: (0, 0)), v_spec,
                  w_spec, v_spec, pl.BlockSpec((H, D), lambda i: (0, 0)),
                  v_spec, v_spec, v_spec],
        out_specs=pl.BlockSpec((block, D), lambda i: (i, 0)),
        out_shape=jax.ShapeDtypeStruct((E, D), _F32),
    )(gp, gq, ea, w0c, b0, w1, b1, w2, b2, g, beta)


def _node_body(x_ref, a0_ref, a1_ref, w0a_ref, w0b_ref, b0_ref, w1_ref,
               b1_ref, w2_ref, b2_ref, g_ref, beta_ref, out_ref):
    xv = x_ref[...]
    agg = a0_ref[...] + a1_ref[...]
    h = _dot(xv, w0a_ref[...]) + _dot(agg, w0b_ref[...]) + b0_ref[...]
    h = jnp.maximum(h, 0.0)
    h = jnp.maximum(_dot(h, w1_ref[...]) + b1_ref[...], 0.0)
    h = _dot(h, w2_ref[...]) + b2_ref[...]
    out_ref[...] = _ln(h, g_ref[...], beta_ref[...]) + xv


def _node_mlp(x, a0, a1, w0a, w0b, b0, w1, b1, w2, b2, g, beta, block):
    grid = N // block
    row_spec = pl.BlockSpec((block, D), lambda i: (i, 0))
    w_spec = pl.BlockSpec((H, H), lambda i: (0, 0))
    v_spec = pl.BlockSpec((1, H), lambda i: (0, 0))
    return pl.pallas_call(
        _node_body,
        grid=(grid,),
        in_specs=[row_spec, row_spec, row_spec,
                  w_spec, w_spec, v_spec, w_spec, v_spec,
                  pl.BlockSpec((H, D), lambda i: (0, 0)), v_spec, v_spec,
                  v_spec],
        out_specs=pl.BlockSpec((block, D), lambda i: (i, 0)),
        out_shape=jax.ShapeDtypeStruct((N, D), _F32),
    )(x, a0, a1, w0a, w0b, b0, w1, b1, w2, b2, g, beta)


_MESH = plsc.VectorSubcoreMesh(core_axis_name="c", subcore_axis_name="s",
                               num_cores=NC, num_subcores=NS)


@functools.partial(
    pl.kernel,
    out_type=[jax.ShapeDtypeStruct((E, H), _F32),
              jax.ShapeDtypeStruct((E, H), _F32)],
    mesh=_MESH,
    scratch_types=[
        pltpu.VMEM((CH,), jnp.int32), pltpu.VMEM((CH,), jnp.int32),
        pltpu.VMEM((CH, H), _F32), pltpu.VMEM((CH, H), _F32),
        pltpu.VMEM((CH,), jnp.int32), pltpu.VMEM((CH,), jnp.int32),
        pltpu.VMEM((CH, H), _F32), pltpu.VMEM((CH, H), _F32),
        pltpu.SemaphoreType.DMA, pltpu.SemaphoreType.DMA,
        pltpu.SemaphoreType.DMA, pltpu.SemaphoreType.DMA,
    ],
)
def _sc_gather(p_hbm, q_hbm, row_hbm, col_hbm, gp_hbm, gq_hbm,
               idxr0, idxc0, bufp0, bufq0, idxr1, idxc1, bufp1, bufq1,
               si0, sg0, si1, sg1):
    wid = lax.axis_index("s") * NC + lax.axis_index("c")
    base = wid * EP
    slots = ((idxr0, idxc0, bufp0, bufq0, si0, sg0),
             (idxr1, idxc1, bufp1, bufq1, si1, sg1))

    def off(t):
        return base + t * CH

    def fire_idx(t, s):
        pltpu.async_copy(row_hbm.at[pl.ds(off(t), CH)], s[0], s[4])
        pltpu.async_copy(col_hbm.at[pl.ds(off(t), CH)], s[1], s[4])

    def wait_idx(t, s):
        pltpu.make_async_copy(row_hbm.at[pl.ds(off(t), CH)], s[0], s[4]).wait()
        pltpu.make_async_copy(col_hbm.at[pl.ds(off(t), CH)], s[1], s[4]).wait()

    def fire_gather(s):
        pltpu.async_copy(p_hbm.at[s[0]], s[2], s[5])
        pltpu.async_copy(q_hbm.at[s[1]], s[3], s[5])

    def wait_gather(s):
        pltpu.make_async_copy(p_hbm.at[s[0]], s[2], s[5]).wait()
        pltpu.make_async_copy(q_hbm.at[s[1]], s[3], s[5]).wait()

    def step(t, b, fire_i, fire_g):
        s, sn = slots[b], slots[1 - b]
        wait_gather(s)
        if fire_i:
            fire_idx(t + 2, s)
        if fire_g:
            wait_idx(t + 1, sn)
            fire_gather(sn)
        pltpu.sync_copy(s[2], gp_hbm.at[pl.ds(off(t), CH)])
        pltpu.sync_copy(s[3], gq_hbm.at[pl.ds(off(t), CH)])

    fire_idx(0, slots[0])
    wait_idx(0, slots[0])
    fire_gather(slots[0])
    fire_idx(1, slots[1])

    def body(g, _):
        step(2 * g, 0, True, True)
        step(2 * g + 1, 1, True, True)
        return _

    lax.fori_loop(0, (NCHUNK - 3) // 2, body, None)
    step(NCHUNK - 3, 0, True, True)
    step(NCHUNK - 2, 1, False, True)
    step(NCHUNK - 1, 0, False, False)


@functools.partial(
    pl.kernel,
    out_type=jax.ShapeDtypeStruct((NC, N, H), _F32),
    mesh=_MESH,
    scratch_types=[
        pltpu.VMEM((1, CH), jnp.int32), pltpu.VMEM((CH, H), _F32),
        pltpu.VMEM((1, CH), jnp.int32), pltpu.VMEM((CH, H), _F32),
        pltpu.SemaphoreType.DMA, pltpu.SemaphoreType.DMA,
        pltpu.VMEM_SHARED((N, H), _F32),
    ],
)
def _sc_scatter(ea_hbm, col_hbm, zeros_hbm, out_hbm,
                idx0, buf0, idx1, buf1, sl0, sl1, acc):
    cid = lax.axis_index("c")
    sid = lax.axis_index("s")
    wid = sid * NC + cid
    base = wid * EP
    slots = ((idx0, buf0, sl0), (idx1, buf1, sl1))

    def zbody(k, _):
        r = sid * RSTEP + k * CH
        pltpu.sync_copy(zeros_hbm.at[pl.ds(r, CH)], buf0)
        pltpu.sync_copy(buf0, acc.at[pl.ds(r, CH)])
        return _

    lax.fori_loop(0, RP // CH, zbody, None)
    plsc.subcore_barrier()

    def off(t):
        return base + t * CH

    def fire_load(t, s):
        pltpu.async_copy(col_hbm.at[pl.ds(off(t), CH)], s[0].at[0], s[2])
        pltpu.async_copy(ea_hbm.at[pl.ds(off(t), CH)], s[1], s[2])

    def wait_load(t, s):
        pltpu.make_async_copy(col_hbm.at[pl.ds(off(t), CH)], s[0].at[0], s[2]).wait()
        pltpu.make_async_copy(ea_hbm.at[pl.ds(off(t), CH)], s[1], s[2]).wait()

    def step(t, b, fire_next):
        s, sn = slots[b], slots[1 - b]
        wait_load(t, s)
        if fire_next:
            fire_load(t + 1, sn)
        pltpu.sync_copy(s[1], acc.at[s[0].at[0]], add=True)

    fire_load(0, slots[0])

    def body(g, _):
        step(2 * g, 0, True)
        step(2 * g + 1, 1, True)
        return _

    lax.fori_loop(0, (NCHUNK - 1) // 2, body, None)
    step(NCHUNK - 1, 0, False)
    plsc.subcore_barrier()

    def obody(k, _):
        r = sid * RSTEP + k * CH
        pltpu.sync_copy(acc.at[pl.ds(r, CH)], buf0)
        pltpu.sync_copy(buf0, out_hbm.at[cid, pl.ds(r, CH)])
        return _

    lax.fori_loop(0, RP // CH, obody, None)


def kernel(x, edge_attr, edge_W0, edge_b0, edge_W1, edge_b1, edge_W2,
           edge_b2, edge_ln_g, edge_ln_b, node_W0, node_b0, node_W1,
           node_b1, node_W2, node_b2, node_ln_g, node_ln_b, edge_index):
    row = edge_index[0]
    col = edge_index[1]
    zeros = jnp.zeros((N, H), _F32)
    nb = edge_W0.shape[0]

    def v(a):
        return a.reshape(1, -1)

    for b in range(nb):
        p, q = _pre(x, edge_W0[b], block=2000)
        gp, gq = _sc_gather(p, q, row, col)
        edge_attr = _edge_mlp(gp, gq, edge_attr, edge_W0[b][2 * D:], v(edge_b0[b]),
                              edge_W1[b], v(edge_b1[b]), edge_W2[b], v(edge_b2[b]),
                              v(edge_ln_g[b]), v(edge_ln_b[b]), block=2560)
        agg = _sc_scatter(edge_attr, col, zeros)
        x = _node_mlp(x, agg[0], agg[1], node_W0[b][:D], node_W0[b][D:],
                      v(node_b0[b]), node_W1[b], v(node_b1[b]), node_W2[b],
                      v(node_b2[b]), v(node_ln_g[b]), v(node_ln_b[b]),
                      block=2000)
    return (x, edge_attr)
